# baseline (speedup 1.0000x reference)
import functools

import jax
import jax.numpy as jnp
import numpy as np
from jax import lax
from jax.experimental import pallas as pl
from jax.experimental.pallas import tpu as pltpu
from jax.experimental.pallas import tpu_sc as plsc

_CROP_H = 7
_CROP_W = 7
_S = _CROP_H * _CROP_W

_B, _C, _H, _W = 4, 256, 224, 224
_HW = _H * _W
_NBOX = 2000
_NP = 2048
_SP = 64

_NC, _NS = 2, 16
_NW = _NC * _NS
_BPW = _NP // _NW
_OSZ = _C * _S

_HWB = 3584


def _transpose_in_body(x_ref, o_ref):
    o_ref[0] = x_ref[0].T


def _make_table(image):
    img3 = image.reshape(_B, _C, _HW)
    t = pl.pallas_call(
        _transpose_in_body,
        grid=(_B, _HW // _HWB),
        in_specs=[pl.BlockSpec((1, _C, _HWB), lambda b, h: (b, 0, h))],
        out_specs=pl.BlockSpec((1, _HWB, _C), lambda b, h: (b, h, 0)),
        out_shape=jax.ShapeDtypeStruct((_B, _HW, _C), jnp.float32),
    )(img3)
    return t.reshape(_B * _HW, _C)


_IW = 208


def _coord_parts(boxes, ss):
    y1 = boxes[:, 0:1]
    x1 = boxes[:, 1:2]
    y2 = boxes[:, 2:3]
    x2 = boxes[:, 3:4]
    i = (ss // _CROP_W).astype(jnp.float32)
    j = (ss % _CROP_W).astype(jnp.float32)
    ys = y1 * (_H - 1) + i * ((y2 - y1) * (_H - 1) / (_CROP_H - 1))
    xs = x1 * (_W - 1) + j * ((x2 - x1) * (_W - 1) / (_CROP_W - 1))
    oy = (ys < 0) | (ys > (_H - 1))
    ox = (xs < 0) | (xs > (_W - 1))
    y0f = jnp.floor(ys)
    x0f = jnp.floor(xs)
    yl = ys - y0f
    xl = xs - x0f
    y0i = jnp.clip(y0f, 0, _H - 1).astype(jnp.int32)
    y1i = jnp.clip(y0f + 1, 0, _H - 1).astype(jnp.int32)
    x0i = jnp.clip(x0f, 0, _W - 1).astype(jnp.int32)
    x1i = jnp.clip(x0f + 1, 0, _W - 1).astype(jnp.int32)
    return y0i, y1i, x0i, x1i, yl, xl, oy, ox


_WW = _S * 64
_NPB = 128


def _coords_body(boxes_ref, bidx_ref, oi_ref, ow_ref):
    boxes = boxes_ref[...]
    base = bidx_ref[:, 0:1] * _HW

    c = lax.broadcasted_iota(jnp.int32, (_NPB, _IW), 1)
    sc = c // 4
    kc = c % 4
    y0i, y1i, x0i, x1i, _, _, _, _ = _coord_parts(boxes, sc)
    yi = jnp.where(kc >= 2, y1i, y0i)
    xi = jnp.where(kc % 2 == 1, x1i, x0i)
    oi_ref[...] = base + yi * _W + xi

    q = lax.broadcasted_iota(jnp.int32, (_NPB, _WW), 1)
    sq = q // 64
    kq = (q % 64) // 16
    _, _, _, _, yl, xl, oy, ox = _coord_parts(boxes, sq)
    vf = jnp.where(~oy & ~ox, 1.0, 0.0).astype(jnp.float32)
    wy = jnp.where(kq >= 2, yl, 1.0 - yl)
    wx = jnp.where(kq % 2 == 1, xl, 1.0 - xl)
    ow_ref[...] = wy * wx * vf


def _coords(boxes_p, bidx_p):
    return pl.pallas_call(
        _coords_body,
        grid=(_NP // _NPB,),
        in_specs=[pl.BlockSpec((_NPB, 4), lambda n: (n, 0)),
                  pl.BlockSpec((_NPB, 128), lambda n: (n, 0))],
        out_specs=(pl.BlockSpec((_NPB, _IW), lambda n: (n, 0)),
                   pl.BlockSpec((_NPB, _WW), lambda n: (n, 0))),
        out_shape=(jax.ShapeDtypeStruct((_NP, _IW), jnp.int32),
                   jax.ShapeDtypeStruct((_NP, _WW), jnp.float32)),
    )(boxes_p, bidx_p)


_MESH = plsc.VectorSubcoreMesh(
    core_axis_name="c", subcore_axis_name="s", num_cores=_NC, num_subcores=_NS
)

_LANE = np.arange(16, dtype=np.int32)


def _sc_gather(table, idx4, w4):
    @functools.partial(
        pl.kernel,
        out_type=jax.ShapeDtypeStruct((_NBOX * _S * 2, 128), jnp.float32),
        mesh=_MESH,
        scratch_types=(
            [pltpu.VMEM((_IW,), jnp.int32) for _ in range(2)]
            + [pltpu.VMEM((_WW,), jnp.float32) for _ in range(2)]
            + [pltpu.VMEM((200, _C), jnp.float32) for _ in range(2)]
            + [pltpu.VMEM((2 * _S, 128), jnp.float32)]
            + [pltpu.SemaphoreType.DMA for _ in range(5)]
        ),
        compiler_params=pltpu.CompilerParams(use_tc_tiling_on_sc=False),
    )
    def k(idx4_h, w4_h, table_h, out_h,
          is0, is1, ws0, ws1, ra, rb, ob,
          isem0, isem1, gsem0, gsem1, osem):
        wid = lax.axis_index("s") * _NC + lax.axis_index("c")
        base_box = wid * _BPW
        rsl = (ra, rb)
        isl = (is0, is1)
        wsl = (ws0, ws1)
        isem = (isem0, isem1)
        gsem = (gsem0, gsem1)

        def issue_ixw(b, slot):
            pltpu.async_copy(idx4_h.at[base_box + b], isl[slot], isem[slot])
            pltpu.async_copy(w4_h.at[base_box + b], wsl[slot], isem[slot])

        def wait_ixw(slot):
            pltpu.make_async_copy(idx4_h.at[0], isl[slot], isem[slot]).wait()
            pltpu.make_async_copy(w4_h.at[0], wsl[slot], isem[slot]).wait()

        def issue_gathers(slot):
            pltpu.async_copy(
                table_h.at[isl[slot].at[pl.ds(0, 104)]],
                rsl[slot].at[pl.ds(0, 104)], gsem[slot])
            pltpu.async_copy(
                table_h.at[isl[slot].at[pl.ds(104, 96)]],
                rsl[slot].at[pl.ds(104, 96)], gsem[slot])

        def wait_gathers(slot):
            pltpu.make_async_copy(
                table_h.at[isl[slot].at[pl.ds(0, 104)]],
                rsl[slot].at[pl.ds(0, 104)], gsem[slot]).wait()
            pltpu.make_async_copy(
                table_h.at[isl[slot].at[pl.ds(104, 96)]],
                rsl[slot].at[pl.ds(104, 96)], gsem[slot]).wait()

        def issue_flush(nb):
            pltpu.async_copy(ob, out_h.at[pl.ds(nb * 2 * _S, 2 * _S)], osem)

        def wait_flush():
            pltpu.make_async_copy(
                ob, out_h.at[pl.ds(0, 2 * _S)], osem).wait()

        def compute_box(slot):
            rr = rsl[slot]
            ww = wsl[slot]

            @plsc.parallel_loop(0, _S, 1, unroll=4)
            def _(s):
                r0 = 4 * s
                w0 = s * 64
                a0 = ww[pl.ds(w0, 16)]
                a1 = ww[pl.ds(w0 + 16, 16)]
                a2 = ww[pl.ds(w0 + 32, 16)]
                a3 = ww[pl.ds(w0 + 48, 16)]
                for cc in range(_C // 16):
                    sl = pl.ds(cc * 16, 16)
                    ob[2 * s + cc // 8, pl.ds((cc % 8) * 16, 16)] = (
                        rr[r0, sl] * a0 + rr[r0 + 1, sl] * a1
                        + rr[r0 + 2, sl] * a2 + rr[r0 + 3, sl] * a3)

        issue_ixw(0, 0)
        wait_ixw(0)
        issue_gathers(0)
        issue_ixw(1, 1)

        def body(b, carry):
            nb = base_box + b

            def half(par):
                @pl.when(b < _BPW - 1)
                def _():
                    wait_ixw(1 - par)
                    issue_gathers(1 - par)
                wait_gathers(par)

                @pl.when((b >= 1) & (nb - 1 < _NBOX))
                def _():
                    wait_flush()
                compute_box(par)

                @pl.when(nb < _NBOX)
                def _():
                    issue_flush(nb)

                @pl.when(b < _BPW - 2)
                def _():
                    issue_ixw(b + 2, par)

            @pl.when(b % 2 == 0)
            def _():
                half(0)

            @pl.when(b % 2 == 1)
            def _():
                half(1)

            return carry

        lax.fori_loop(0, _BPW, body, 0)

        @pl.when(base_box + _BPW - 1 < _NBOX)
        def _():
            wait_flush()

    return k(idx4, w4, table)


_NBLK = 8
_EYE = np.eye(_S, dtype=np.float32)


def _final_body(x_ref, eye_ref, o_ref):
    eye = eye_ref[...]
    for n in range(_NBLK):
        o_ref[n] = lax.dot_general(
            x_ref[n], eye, (((0,), (0,)), ((), ())),
            precision=lax.Precision.HIGHEST,
            preferred_element_type=jnp.float32)


def _final(out3):
    return pl.pallas_call(
        _final_body,
        grid=(_NBOX // _NBLK,),
        in_specs=[pl.BlockSpec((_NBLK, _S, _C), lambda n: (n, 0, 0)),
                  pl.BlockSpec((_S, _S), lambda n: (0, 0))],
        out_specs=pl.BlockSpec((_NBLK, _C, _S), lambda n: (n, 0, 0)),
        out_shape=jax.ShapeDtypeStruct((_NBOX, _C, _S), jnp.float32),
    )(out3, jnp.asarray(_EYE))


def kernel(image, boxes, box_indices):
    table = _make_table(image)
    boxes_p = jnp.pad(boxes, ((0, _NP - _NBOX), (0, 0)))
    bidx_p = jnp.broadcast_to(
        jnp.pad(box_indices.astype(jnp.int32), (0, _NP - _NBOX))[:, None],
        (_NP, 128),
    )
    idx4, w4 = _coords(boxes_p, bidx_p)
    out_sc = _sc_gather(table, idx4, w4)
    out5 = out_sc.reshape(_NBOX, _S, 2, 128)
    out = jnp.transpose(out5, (0, 2, 3, 1))
    return out.reshape(_NBOX, _C, _CROP_H, _CROP_W)


# speedup vs baseline: 1.0173x; 1.0173x over previous
import functools

import jax
import jax.numpy as jnp
import numpy as np
from jax import lax
from jax.experimental import pallas as pl
from jax.experimental.pallas import tpu as pltpu
from jax.experimental.pallas import tpu_sc as plsc

_CROP_H = 7
_CROP_W = 7
_S = _CROP_H * _CROP_W

_B, _C, _H, _W = 4, 256, 224, 224
_HW = _H * _W
_NBOX = 2000
_NP = 2048
_SP = 64

_NC, _NS = 2, 16
_NW = _NC * _NS
_BPW = _NP // _NW
_OSZ = _C * _S

_HWB = 3584


def _transpose_in_body(x_ref, o_ref):
    o_ref[0] = x_ref[0].T


def _make_table(image):
    img3 = image.reshape(_B, _C, _HW)
    t = pl.pallas_call(
        _transpose_in_body,
        grid=(_B, _HW // _HWB),
        in_specs=[pl.BlockSpec((1, _C, _HWB), lambda b, h: (b, 0, h))],
        out_specs=pl.BlockSpec((1, _HWB, _C), lambda b, h: (b, h, 0)),
        out_shape=jax.ShapeDtypeStruct((_B, _HW, _C), jnp.float32),
    )(img3)
    return t.reshape(_B * _HW, _C)


_IW = 208


def _coord_parts(boxes, ss):
    y1 = boxes[:, 0:1]
    x1 = boxes[:, 1:2]
    y2 = boxes[:, 2:3]
    x2 = boxes[:, 3:4]
    i = (ss // _CROP_W).astype(jnp.float32)
    j = (ss % _CROP_W).astype(jnp.float32)
    ys = y1 * (_H - 1) + i * ((y2 - y1) * (_H - 1) / (_CROP_H - 1))
    xs = x1 * (_W - 1) + j * ((x2 - x1) * (_W - 1) / (_CROP_W - 1))
    oy = (ys < 0) | (ys > (_H - 1))
    ox = (xs < 0) | (xs > (_W - 1))
    y0f = jnp.floor(ys)
    x0f = jnp.floor(xs)
    yl = ys - y0f
    xl = xs - x0f
    y0i = jnp.clip(y0f, 0, _H - 1).astype(jnp.int32)
    y1i = jnp.clip(y0f + 1, 0, _H - 1).astype(jnp.int32)
    x0i = jnp.clip(x0f, 0, _W - 1).astype(jnp.int32)
    x1i = jnp.clip(x0f + 1, 0, _W - 1).astype(jnp.int32)
    return y0i, y1i, x0i, x1i, yl, xl, oy, ox


_WW = _S * 64
_NPB = 128


def _coords_body(boxes_ref, bidx_ref, oi_ref, ow_ref):
    boxes = boxes_ref[...]
    base = bidx_ref[:, 0:1] * _HW

    c = lax.broadcasted_iota(jnp.int32, (_NPB, _IW), 1)
    sc = c // 4
    kc = c % 4
    y0i, y1i, x0i, x1i, _, _, _, _ = _coord_parts(boxes, sc)
    yi = jnp.where(kc >= 2, y1i, y0i)
    xi = jnp.where(kc % 2 == 1, x1i, x0i)
    oi_ref[...] = base + yi * _W + xi

    q = lax.broadcasted_iota(jnp.int32, (_NPB, _WW), 1)
    sq = q // 64
    kq = (q % 64) // 16
    _, _, _, _, yl, xl, oy, ox = _coord_parts(boxes, sq)
    vf = jnp.where(~oy & ~ox, 1.0, 0.0).astype(jnp.float32)
    wy = jnp.where(kq >= 2, yl, 1.0 - yl)
    wx = jnp.where(kq % 2 == 1, xl, 1.0 - xl)
    ow_ref[...] = wy * wx * vf


def _coords(boxes_p, bidx_p):
    return pl.pallas_call(
        _coords_body,
        grid=(_NP // _NPB,),
        in_specs=[pl.BlockSpec((_NPB, 4), lambda n: (n, 0)),
                  pl.BlockSpec((_NPB, 128), lambda n: (n, 0))],
        out_specs=(pl.BlockSpec((_NPB, _IW), lambda n: (n, 0)),
                   pl.BlockSpec((_NPB, _WW), lambda n: (n, 0))),
        out_shape=(jax.ShapeDtypeStruct((_NP, _IW), jnp.int32),
                   jax.ShapeDtypeStruct((_NP, _WW), jnp.float32)),
    )(boxes_p, bidx_p)


_MESH = plsc.VectorSubcoreMesh(
    core_axis_name="c", subcore_axis_name="s", num_cores=_NC, num_subcores=_NS
)

_LANE = np.arange(16, dtype=np.int32)


def _sc_gather(table, idx4, w4):
    @functools.partial(
        pl.kernel,
        out_type=jax.ShapeDtypeStruct((_NBOX * _S, _C), jnp.float32),
        mesh=_MESH,
        scratch_types=(
            [pltpu.VMEM((_IW,), jnp.int32) for _ in range(2)]
            + [pltpu.VMEM((_WW,), jnp.float32) for _ in range(2)]
            + [pltpu.VMEM((200, _C), jnp.float32) for _ in range(2)]
            + [pltpu.VMEM((_S, _C), jnp.float32)]
            + [pltpu.SemaphoreType.DMA for _ in range(5)]
        ),
        compiler_params=pltpu.CompilerParams(use_tc_tiling_on_sc=False),
    )
    def k(idx4_h, w4_h, table_h, out_h,
          is0, is1, ws0, ws1, ra, rb, ob,
          isem0, isem1, gsem0, gsem1, osem):
        wid = lax.axis_index("s") * _NC + lax.axis_index("c")
        base_box = wid * _BPW
        rsl = (ra, rb)
        isl = (is0, is1)
        wsl = (ws0, ws1)
        isem = (isem0, isem1)
        gsem = (gsem0, gsem1)

        def issue_ixw(b, slot):
            pltpu.async_copy(idx4_h.at[base_box + b], isl[slot], isem[slot])
            pltpu.async_copy(w4_h.at[base_box + b], wsl[slot], isem[slot])

        def wait_ixw(slot):
            pltpu.make_async_copy(idx4_h.at[0], isl[slot], isem[slot]).wait()
            pltpu.make_async_copy(w4_h.at[0], wsl[slot], isem[slot]).wait()

        def issue_gathers(slot):
            pltpu.async_copy(
                table_h.at[isl[slot].at[pl.ds(0, 104)]],
                rsl[slot].at[pl.ds(0, 104)], gsem[slot])
            pltpu.async_copy(
                table_h.at[isl[slot].at[pl.ds(104, 96)]],
                rsl[slot].at[pl.ds(104, 96)], gsem[slot])

        def wait_gathers(slot):
            pltpu.make_async_copy(
                table_h.at[isl[slot].at[pl.ds(0, 104)]],
                rsl[slot].at[pl.ds(0, 104)], gsem[slot]).wait()
            pltpu.make_async_copy(
                table_h.at[isl[slot].at[pl.ds(104, 96)]],
                rsl[slot].at[pl.ds(104, 96)], gsem[slot]).wait()

        def issue_flush(nb):
            pltpu.async_copy(ob, out_h.at[pl.ds(nb * _S, _S)], osem)

        def wait_flush():
            pltpu.make_async_copy(
                ob, out_h.at[pl.ds(0, _S)], osem).wait()

        def compute_box(slot):
            rr = rsl[slot]
            ww = wsl[slot]

            @plsc.parallel_loop(0, _S, 1, unroll=4)
            def _(s):
                r0 = 4 * s
                w0 = s * 64
                a0 = ww[pl.ds(w0, 16)]
                a1 = ww[pl.ds(w0 + 16, 16)]
                a2 = ww[pl.ds(w0 + 32, 16)]
                a3 = ww[pl.ds(w0 + 48, 16)]
                for cc in range(_C // 16):
                    sl = pl.ds(cc * 16, 16)
                    ob[s, sl] = (
                        rr[r0, sl] * a0 + rr[r0 + 1, sl] * a1
                        + rr[r0 + 2, sl] * a2 + rr[r0 + 3, sl] * a3)

        issue_ixw(0, 0)
        wait_ixw(0)
        issue_gathers(0)
        issue_ixw(1, 1)

        def body(b, carry):
            nb = base_box + b

            def half(par):
                @pl.when(b < _BPW - 1)
                def _():
                    wait_ixw(1 - par)
                    issue_gathers(1 - par)
                wait_gathers(par)

                @pl.when((b >= 1) & (nb - 1 < _NBOX))
                def _():
                    wait_flush()
                compute_box(par)

                @pl.when(nb < _NBOX)
                def _():
                    issue_flush(nb)

                @pl.when(b < _BPW - 2)
                def _():
                    issue_ixw(b + 2, par)

            @pl.when(b % 2 == 0)
            def _():
                half(0)

            @pl.when(b % 2 == 1)
            def _():
                half(1)

            return carry

        lax.fori_loop(0, _BPW, body, 0)

        @pl.when(base_box + _BPW - 1 < _NBOX)
        def _():
            wait_flush()

    return k(idx4, w4, table)


_NBLK = 8
_EYE = np.eye(_S, dtype=np.float32)


def _final_body(x_ref, eye_ref, o_ref):
    eye = eye_ref[...]
    for n in range(_NBLK):
        o_ref[n] = lax.dot_general(
            x_ref[n], eye, (((0,), (0,)), ((), ())),
            precision=lax.Precision.HIGHEST,
            preferred_element_type=jnp.float32)


def _final(out3):
    return pl.pallas_call(
        _final_body,
        grid=(_NBOX // _NBLK,),
        in_specs=[pl.BlockSpec((_NBLK, _S, _C), lambda n: (n, 0, 0)),
                  pl.BlockSpec((_S, _S), lambda n: (0, 0))],
        out_specs=pl.BlockSpec((_NBLK, _C, _S), lambda n: (n, 0, 0)),
        out_shape=jax.ShapeDtypeStruct((_NBOX, _C, _S), jnp.float32),
    )(out3, jnp.asarray(_EYE))


def kernel(image, boxes, box_indices):
    table = _make_table(image)
    boxes_p = jnp.pad(boxes, ((0, _NP - _NBOX), (0, 0)))
    bidx_p = jnp.broadcast_to(
        jnp.pad(box_indices.astype(jnp.int32), (0, _NP - _NBOX))[:, None],
        (_NP, 128),
    )
    idx4, w4 = _coords(boxes_p, bidx_p)
    out_sc = _sc_gather(table, idx4, w4)
    out3 = out_sc.reshape(_NBOX, _CROP_H, _CROP_W, _C)
    return jnp.transpose(out3, (0, 3, 1, 2))


# speedup vs baseline: 1.0875x; 1.0690x over previous
import functools

import jax
import jax.numpy as jnp
import numpy as np
from jax import lax
from jax.experimental import pallas as pl
from jax.experimental.pallas import tpu as pltpu
from jax.experimental.pallas import tpu_sc as plsc

_CROP_H = 7
_CROP_W = 7
_S = _CROP_H * _CROP_W

_B, _C, _H, _W = 4, 256, 224, 224
_HW = _H * _W
_NBOX = 2000
_NP = 2048
_SP = 64

_NC, _NS = 2, 16
_NW = _NC * _NS
_BPW = _NP // _NW
_OSZ = _C * _S

_HWB = 3584


def _transpose_in_body(x_ref, o_ref):
    o_ref[0] = x_ref[0].T


def _make_table(image):
    img3 = image.reshape(_B, _C, _HW)
    t = pl.pallas_call(
        _transpose_in_body,
        grid=(_B, _HW // _HWB),
        in_specs=[pl.BlockSpec((1, _C, _HWB), lambda b, h: (b, 0, h))],
        out_specs=pl.BlockSpec((1, _HWB, _C), lambda b, h: (b, h, 0)),
        out_shape=jax.ShapeDtypeStruct((_B, _HW, _C), jnp.float32),
    )(img3)
    return t.reshape(_B * _HW, _C)


_IW = 208


def _coord_parts(boxes, ss):
    y1 = boxes[:, 0:1]
    x1 = boxes[:, 1:2]
    y2 = boxes[:, 2:3]
    x2 = boxes[:, 3:4]
    i = (ss // _CROP_W).astype(jnp.float32)
    j = (ss % _CROP_W).astype(jnp.float32)
    ys = y1 * (_H - 1) + i * ((y2 - y1) * (_H - 1) / (_CROP_H - 1))
    xs = x1 * (_W - 1) + j * ((x2 - x1) * (_W - 1) / (_CROP_W - 1))
    oy = (ys < 0) | (ys > (_H - 1))
    ox = (xs < 0) | (xs > (_W - 1))
    y0f = jnp.floor(ys)
    x0f = jnp.floor(xs)
    yl = ys - y0f
    xl = xs - x0f
    y0i = jnp.clip(y0f, 0, _H - 1).astype(jnp.int32)
    y1i = jnp.clip(y0f + 1, 0, _H - 1).astype(jnp.int32)
    x0i = jnp.clip(x0f, 0, _W - 1).astype(jnp.int32)
    x1i = jnp.clip(x0f + 1, 0, _W - 1).astype(jnp.int32)
    return y0i, y1i, x0i, x1i, yl, xl, oy, ox


_IR = 8
_WR = 32
_IBLK = 1024
_WBLK = 4096


def _coords_idx_body(boxes_ref, bidx_ref, oi_ref):
    boxes = boxes_ref[...]
    r = lax.broadcasted_iota(jnp.int32, (_IBLK, 128), 0)
    l = lax.broadcasted_iota(jnp.int32, (_IBLK, 128), 1)
    c = (r % _IR) * 128 + l
    sc = c // 4
    kc = c % 4
    y0i, y1i, x0i, x1i, _, _, _, _ = _coord_parts(boxes, sc)
    yi = jnp.where(kc >= 2, y1i, y0i)
    xi = jnp.where(kc % 2 == 1, x1i, x0i)
    base = bidx_ref[:, 0:1] * _HW
    oi_ref[...] = base + yi * _W + xi


def _coords_w_body(boxes_ref, ow_ref):
    boxes = boxes_ref[...]
    r = lax.broadcasted_iota(jnp.int32, (_WBLK, 128), 0)
    l = lax.broadcasted_iota(jnp.int32, (_WBLK, 128), 1)
    q = (r % _WR) * 128 + l
    sq = q // 64
    kq = (q % 64) // 16
    _, _, _, _, yl, xl, oy, ox = _coord_parts(boxes, sq)
    vf = jnp.where(~oy & ~ox & (sq < _S), 1.0, 0.0).astype(jnp.float32)
    wy = jnp.where(kq >= 2, yl, 1.0 - yl)
    wx = jnp.where(kq % 2 == 1, xl, 1.0 - xl)
    ow_ref[...] = wy * wx * vf


def _coords(boxes_p, bidx_p):
    boxes8 = jnp.repeat(boxes_p, _IR, axis=0)
    bidx8 = jnp.repeat(bidx_p, _IR, axis=0)
    boxes32 = jnp.repeat(boxes_p, _WR, axis=0)
    idx2 = pl.pallas_call(
        _coords_idx_body,
        grid=(_NP * _IR // _IBLK,),
        in_specs=[pl.BlockSpec((_IBLK, 4), lambda n: (n, 0)),
                  pl.BlockSpec((_IBLK, 128), lambda n: (n, 0))],
        out_specs=pl.BlockSpec((_IBLK, 128), lambda n: (n, 0)),
        out_shape=jax.ShapeDtypeStruct((_NP * _IR, 128), jnp.int32),
    )(boxes8, bidx8)
    w2 = pl.pallas_call(
        _coords_w_body,
        grid=(_NP * _WR // _WBLK,),
        in_specs=[pl.BlockSpec((_WBLK, 4), lambda n: (n, 0))],
        out_specs=pl.BlockSpec((_WBLK, 128), lambda n: (n, 0)),
        out_shape=jax.ShapeDtypeStruct((_NP * _WR, 128), jnp.float32),
    )(boxes32)
    return idx2, w2


_MESH = plsc.VectorSubcoreMesh(
    core_axis_name="c", subcore_axis_name="s", num_cores=_NC, num_subcores=_NS
)

_LANE = np.arange(16, dtype=np.int32)


def _sc_gather(table, idx4, w4):
    @functools.partial(
        pl.kernel,
        out_type=jax.ShapeDtypeStruct((_NBOX, 56, _C), jnp.float32),
        mesh=_MESH,
        scratch_types=(
            [pltpu.VMEM((_IR, 128), jnp.int32) for _ in range(2)]
            + [pltpu.VMEM((_WR, 128), jnp.float32) for _ in range(2)]
            + [pltpu.VMEM((200, _C), jnp.float32) for _ in range(2)]
            + [pltpu.VMEM((56, _C), jnp.float32)]
            + [pltpu.SemaphoreType.DMA for _ in range(5)]
        ),
    )
    def k(idx4_h, w4_h, table_h, out_h,
          is0, is1, ws0, ws1, ra, rb, ob,
          isem0, isem1, gsem0, gsem1, osem):
        wid = lax.axis_index("s") * _NC + lax.axis_index("c")
        base_box = wid * _BPW
        rsl = (ra, rb)
        isl = (is0, is1)
        wsl = (ws0, ws1)
        isem = (isem0, isem1)
        gsem = (gsem0, gsem1)

        def issue_ixw(b, slot):
            nb = base_box + b
            pltpu.async_copy(
                idx4_h.at[pl.ds(nb * _IR, _IR)], isl[slot], isem[slot])
            pltpu.async_copy(
                w4_h.at[pl.ds(nb * _WR, _WR)], wsl[slot], isem[slot])

        def wait_ixw(slot):
            pltpu.make_async_copy(
                idx4_h.at[pl.ds(0, _IR)], isl[slot], isem[slot]).wait()
            pltpu.make_async_copy(
                w4_h.at[pl.ds(0, _WR)], wsl[slot], isem[slot]).wait()

        def issue_gathers(slot):
            pltpu.async_copy(
                table_h.at[isl[slot].at[0]],
                rsl[slot].at[pl.ds(0, 128)], gsem[slot])
            pltpu.async_copy(
                table_h.at[isl[slot].at[1, pl.ds(0, 72)]],
                rsl[slot].at[pl.ds(128, 72)], gsem[slot])

        def wait_gathers(slot):
            pltpu.make_async_copy(
                table_h.at[isl[slot].at[0]],
                rsl[slot].at[pl.ds(0, 128)], gsem[slot]).wait()
            pltpu.make_async_copy(
                table_h.at[isl[slot].at[1, pl.ds(0, 72)]],
                rsl[slot].at[pl.ds(128, 72)], gsem[slot]).wait()

        def issue_flush(nb):
            pltpu.async_copy(ob, out_h.at[nb], osem)

        def wait_flush():
            pltpu.make_async_copy(ob, out_h.at[0], osem).wait()

        def compute_box(slot):
            rr = rsl[slot]
            ww = wsl[slot]

            @plsc.parallel_loop(0, _S, 1, unroll=4)
            def _(s):
                r0 = 4 * s
                wr = s // 2
                wc = (s % 2) * 64
                a0 = ww[wr, pl.ds(wc, 16)]
                a1 = ww[wr, pl.ds(wc + 16, 16)]
                a2 = ww[wr, pl.ds(wc + 32, 16)]
                a3 = ww[wr, pl.ds(wc + 48, 16)]
                for cc in range(_C // 16):
                    sl = pl.ds(cc * 16, 16)
                    ob[s, sl] = (
                        rr[r0, sl] * a0 + rr[r0 + 1, sl] * a1
                        + rr[r0 + 2, sl] * a2 + rr[r0 + 3, sl] * a3)

        issue_ixw(0, 0)
        wait_ixw(0)
        issue_gathers(0)
        issue_ixw(1, 1)

        def body(b, carry):
            nb = base_box + b

            def half(par):
                @pl.when(b < _BPW - 1)
                def _():
                    wait_ixw(1 - par)
                    issue_gathers(1 - par)
                wait_gathers(par)

                @pl.when((b >= 1) & (nb - 1 < _NBOX))
                def _():
                    wait_flush()
                compute_box(par)

                @pl.when(nb < _NBOX)
                def _():
                    issue_flush(nb)

                @pl.when(b < _BPW - 2)
                def _():
                    issue_ixw(b + 2, par)

            @pl.when(b % 2 == 0)
            def _():
                half(0)

            @pl.when(b % 2 == 1)
            def _():
                half(1)

            return carry

        lax.fori_loop(0, _BPW, body, 0)

        @pl.when(base_box + _BPW - 1 < _NBOX)
        def _():
            wait_flush()

    return k(idx4, w4, table)


_NBLK = 8
_EYE = np.eye(_S, dtype=np.float32)


def _final_body(x_ref, eye_ref, o_ref):
    eye = eye_ref[...]
    for n in range(_NBLK):
        o_ref[n] = lax.dot_general(
            x_ref[n], eye, (((0,), (0,)), ((), ())),
            precision=lax.Precision.HIGHEST,
            preferred_element_type=jnp.float32)


def _final(out3):
    return pl.pallas_call(
        _final_body,
        grid=(_NBOX // _NBLK,),
        in_specs=[pl.BlockSpec((_NBLK, _S, _C), lambda n: (n, 0, 0)),
                  pl.BlockSpec((_S, _S), lambda n: (0, 0))],
        out_specs=pl.BlockSpec((_NBLK, _C, _S), lambda n: (n, 0, 0)),
        out_shape=jax.ShapeDtypeStruct((_NBOX, _C, _S), jnp.float32),
    )(out3, jnp.asarray(_EYE))


def kernel(image, boxes, box_indices):
    table = _make_table(image)
    boxes_p = jnp.pad(boxes, ((0, _NP - _NBOX), (0, 0)))
    bidx_p = jnp.broadcast_to(
        jnp.pad(box_indices.astype(jnp.int32), (0, _NP - _NBOX))[:, None],
        (_NP, 128),
    )
    idx4, w4 = _coords(boxes_p, bidx_p)
    out_sc = _sc_gather(table, idx4, w4)
    out3 = out_sc[:, :_S].reshape(_NBOX, _CROP_H, _CROP_W, _C)
    return jnp.transpose(out3, (0, 3, 1, 2))
